# baseline (speedup 1.0000x reference)
import functools

import jax
import jax.numpy as jnp
from jax import lax
from jax.experimental import pallas as pl
from jax.experimental.pallas import tpu as pltpu
from jax.experimental.pallas import tpu_sc as plsc

NUM_USERS = 100000
NUM_ITEMS = 100000
EMBED_DIM = 64
BATCH = 16384

NUM_CORES = 2
NUM_SUBCORES = 16
NW = NUM_CORES * NUM_SUBCORES
BPW = BATCH // NW
CHUNK = 128
NCHUNK = BPW // CHUNK
LANES = 16
NGROUP = BPW // LANES


def _dot_body(users_hbm, items_hbm, ut_hbm, it_hbm, out_hbm,
              uidx, iidx, urows, irows, tbuf, outv, sem):
    wid = lax.axis_index("s") * NUM_CORES + lax.axis_index("c")

    pltpu.sync_copy(users_hbm.at[wid], uidx)
    pltpu.sync_copy(items_hbm.at[wid], iidx)

    copies = []
    for ch in range(NCHUNK):
        dst = pl.ds(ch * CHUNK, CHUNK)
        copies.append(pltpu.async_copy(ut_hbm.at[uidx.at[ch]], urows.at[dst], sem))
        copies.append(pltpu.async_copy(it_hbm.at[iidx.at[ch]], irows.at[dst], sem))
    for cp in copies:
        cp.wait()

    col = lax.iota(jnp.int32, LANES) * LANES

    def group(g, carry):
        for b in range(LANES):
            row = g * LANES + b
            p = jnp.zeros((LANES,), jnp.float32)
            for k in range(EMBED_DIM // (2 * LANES)):
                u = urows[row, pl.ds(k * 2 * LANES, 2 * LANES)]
                v = irows[row, pl.ds(k * 2 * LANES, 2 * LANES)]
                u0, u1 = plsc.unpack(u, format=plsc.PackFormat.INTERLEAVED)
                v0, v1 = plsc.unpack(v, format=plsc.PackFormat.INTERLEAVED)
                p = p + u0 * v0 + u1 * v1
            plsc.store_scatter(tbuf, [col + b], p)
        acc = jnp.zeros((LANES,), jnp.float32)
        for r in range(LANES):
            acc = acc + tbuf[pl.ds(r * LANES, LANES)]
        outv[pl.ds(g * LANES, LANES)] = acc
        return carry

    lax.fori_loop(0, NGROUP, group, 0)

    pltpu.sync_copy(outv, out_hbm.at[wid])


@jax.jit
def kernel(users, items, user_table, item_table):
    mesh = plsc.VectorSubcoreMesh(core_axis_name="c", subcore_axis_name="s",
                                  num_cores=NUM_CORES, num_subcores=NUM_SUBCORES)
    run = functools.partial(
        pl.kernel,
        out_type=jax.ShapeDtypeStruct((NW, BPW), jnp.float32),
        mesh=mesh,
        scratch_types=[
            pltpu.VMEM((NCHUNK, CHUNK), jnp.int32),
            pltpu.VMEM((NCHUNK, CHUNK), jnp.int32),
            pltpu.VMEM((BPW, EMBED_DIM), jnp.bfloat16),
            pltpu.VMEM((BPW, EMBED_DIM), jnp.bfloat16),
            pltpu.VMEM((LANES * LANES,), jnp.float32),
            pltpu.VMEM((BPW,), jnp.float32),
            pltpu.SemaphoreType.DMA,
        ],
        compiler_params=pltpu.CompilerParams(needs_layout_passes=False,
                                             use_tc_tiling_on_sc=False),
    )(_dot_body)
    out = run(users.reshape(NW, NCHUNK, CHUNK).astype(jnp.int32),
              items.reshape(NW, NCHUNK, CHUNK).astype(jnp.int32),
              user_table.astype(jnp.bfloat16), item_table.astype(jnp.bfloat16))
    return out.reshape(BATCH)


# speedup vs baseline: 1.7568x; 1.7568x over previous
import functools

import jax
import jax.numpy as jnp
from jax import lax
from jax.experimental import pallas as pl
from jax.experimental.pallas import tpu as pltpu
from jax.experimental.pallas import tpu_sc as plsc

NUM_USERS = 100000
NUM_ITEMS = 100000
EMBED_DIM = 64
BATCH = 16384

NUM_CORES = 2
NUM_SUBCORES = 16
NW = NUM_CORES * NUM_SUBCORES
BPW = BATCH // NW
LANES = 16
HALF = BPW // 2
NGROUP = HALF // LANES


def _dot_body(users_hbm, items_hbm, ut_hbm, it_hbm, out_hbm,
              uidx, iidx, urows, irows, tbuf, outv, sem0, sem1):
    wid = lax.axis_index("s") * NUM_CORES + lax.axis_index("c")
    base = wid * BPW

    pltpu.sync_copy(users_hbm.at[pl.ds(base, BPW)], uidx)
    pltpu.sync_copy(items_hbm.at[pl.ds(base, BPW)], iidx)

    col = lax.iota(jnp.int32, LANES) * LANES

    def issue(h0, g, sem):
        i0 = g * LANES
        uvec = uidx[pl.ds(h0 + i0, LANES)]
        ivec = iidx[pl.ds(h0 + i0, LANES)]
        for j in range(LANES):
            pltpu.async_copy(ut_hbm.at[uvec[j]], urows.at[i0 + j], sem)
            pltpu.async_copy(it_hbm.at[ivec[j]], irows.at[i0 + j], sem)

    def drain_group(sem):
        for j in range(LANES):
            pltpu.make_async_copy(ut_hbm.at[0], urows.at[j], sem).wait()
            pltpu.make_async_copy(it_hbm.at[0], irows.at[j], sem).wait()

    def compute_group(h0, g):
        for b in range(LANES):
            row = g * LANES + b
            p = jnp.zeros((LANES,), jnp.float32)
            for k in range(EMBED_DIM // LANES):
                u = urows[row, pl.ds(k * LANES, LANES)]
                v = irows[row, pl.ds(k * LANES, LANES)]
                p = p + u * v
            plsc.store_scatter(tbuf, [col + b], p)
        acc = jnp.zeros((LANES,), jnp.float32)
        for r in range(LANES):
            acc = acc + tbuf[pl.ds(r * LANES, LANES)]
        outv[pl.ds(h0 + g * LANES, LANES)] = acc

    def half(h, carry):
        h0 = h * HALF
        issue(h0, 0, sem0)

        def step(p, carry):
            issue(h0, 2 * p + 1, sem1)
            drain_group(sem0)
            compute_group(h0, 2 * p)

            @pl.when(2 * p + 2 < NGROUP)
            def _():
                issue(h0, 2 * p + 2, sem0)

            drain_group(sem1)
            compute_group(h0, 2 * p + 1)
            return carry

        lax.fori_loop(0, NGROUP // 2, step, 0)
        return carry

    lax.fori_loop(0, 2, half, 0)

    pltpu.sync_copy(outv, out_hbm.at[pl.ds(base, BPW)])


@jax.jit
def kernel(users, items, user_table, item_table):
    mesh = plsc.VectorSubcoreMesh(core_axis_name="c", subcore_axis_name="s",
                                  num_cores=NUM_CORES, num_subcores=NUM_SUBCORES)
    run = functools.partial(
        pl.kernel,
        out_type=jax.ShapeDtypeStruct((BATCH,), jnp.float32),
        mesh=mesh,
        scratch_types=[
            pltpu.VMEM((BPW,), jnp.int32),
            pltpu.VMEM((BPW,), jnp.int32),
            pltpu.VMEM((HALF, EMBED_DIM), jnp.float32),
            pltpu.VMEM((HALF, EMBED_DIM), jnp.float32),
            pltpu.VMEM((LANES * LANES,), jnp.float32),
            pltpu.VMEM((BPW,), jnp.float32),
            pltpu.SemaphoreType.DMA,
            pltpu.SemaphoreType.DMA,
        ],
        compiler_params=pltpu.CompilerParams(needs_layout_passes=False),
    )(_dot_body)
    return run(users.astype(jnp.int32), items.astype(jnp.int32),
               user_table, item_table)
